# baseline (speedup 1.0000x reference)
import functools

import jax
import jax.numpy as jnp
from jax import lax
from jax.experimental import pallas as pl
from jax.experimental.pallas import tpu as pltpu
from jax.experimental.pallas import tpu_sc as plsc

EMBED = 64
BATCH = 16384
L = 16
NC, NS = 2, 16
NW = NC * NS
BPW = BATCH // NW
VOCAB_BLOCKS = 125000
NHALF = 8

_mesh = plsc.VectorSubcoreMesh(core_axis_name="c", subcore_axis_name="s")


@functools.partial(
    pl.kernel,
    out_type=jax.ShapeDtypeStruct((BATCH,), jnp.float32),
    mesh=_mesh,
    compiler_params=pltpu.CompilerParams(
        use_tc_tiling_on_sc=True, needs_layout_passes=False
    ),
    scratch_types=[
        pltpu.VMEM((BPW,), jnp.int32),
        pltpu.VMEM((BPW,), jnp.int32),
        pltpu.VMEM((2 * L, EMBED), jnp.float32),
        pltpu.VMEM((2 * L, EMBED), jnp.float32),
        pltpu.VMEM((BPW,), jnp.float32),
    ]
    + [pltpu.SemaphoreType.DMA] * NHALF
    + [pltpu.SemaphoreType.DMA] * NHALF,
)
def _skipgram_sc(cw_hbm, tw_hbm, ct_hbm, tt_hbm, out_hbm,
                 cidx, tidx, cr, tr, outv, *sems):
    csems = sems[:NHALF]
    tsems = sems[NHALF:]
    wid = lax.axis_index("s") * NC + lax.axis_index("c")
    base = wid * BPW

    pltpu.sync_copy(cw_hbm.at[pl.ds(base, BPW)], cidx)
    pltpu.sync_copy(tw_hbm.at[pl.ds(base, BPW)], tidx)

    def fire(c_row, t_row, slot, k):
        pltpu.async_copy(ct_hbm.at[c_row >> 3, c_row & 7],
                         cr.at[slot], csems[k])
        pltpu.async_copy(tt_hbm.at[t_row >> 3, t_row & 7],
                         tr.at[slot], tsems[k])

    def drain(slot, k):
        pltpu.make_async_copy(ct_hbm.at[0, 0], cr.at[slot], csems[k]).wait()
        pltpu.make_async_copy(tt_hbm.at[0, 0], tr.at[slot], tsems[k]).wait()

    lane = lax.iota(jnp.int32, L)

    civ0 = cidx[pl.ds(0, L)]
    tiv0 = tidx[pl.ds(0, L)]
    for k in range(NHALF):
        fire(civ0[k], tiv0[k], k, k)

    def compute_block(h):
        rows = lane + h * L
        acc = jnp.zeros((L,), jnp.float32)
        for j in range(EMBED):
            cols = jnp.zeros((L,), jnp.int32) + j
            acc = acc + (plsc.load_gather(cr, [rows, cols])
                         * plsc.load_gather(tr, [rows, cols]))
        return acc

    def block_steps(g, civ, tiv, nciv, ntiv, last):
        h = g & 1
        s0 = h * L
        for k in range(NHALF):
            drain(s0 + k, k)
        for k in range(NHALF, L):
            fire(civ[k], tiv[k], s0 + k, k - NHALF)
        for k in range(NHALF, L):
            drain(s0 + k, k - NHALF)
        if not last:
            t0 = (1 - h) * L
            for k in range(NHALF):
                fire(nciv[k], ntiv[k], t0 + k, k)
        acc = compute_block(h)
        outv[pl.ds(g * L, L)] = acc

    def block_body(g, carry):
        civ = cidx[pl.ds(g * L, L)]
        tiv = tidx[pl.ds(g * L, L)]
        nciv = cidx[pl.ds((g + 1) * L, L)]
        ntiv = tidx[pl.ds((g + 1) * L, L)]
        block_steps(g, civ, tiv, nciv, ntiv, last=False)
        return carry

    n_blocks = BPW // L
    lax.fori_loop(0, n_blocks - 1, block_body, 0)

    g_last = n_blocks - 1
    civ = cidx[pl.ds(g_last * L, L)]
    tiv = tidx[pl.ds(g_last * L, L)]
    block_steps(g_last, civ, tiv, civ, tiv, last=True)

    pltpu.sync_copy(outv, out_hbm.at[pl.ds(base, BPW)])


def kernel(center_words, target_words, center_table, target_table):
    return _skipgram_sc(
        center_words.astype(jnp.int32),
        target_words.astype(jnp.int32),
        center_table.reshape(VOCAB_BLOCKS, 8, EMBED),
        target_table.reshape(VOCAB_BLOCKS, 8, EMBED),
    )


# speedup vs baseline: 1.0334x; 1.0334x over previous
import functools

import jax
import jax.numpy as jnp
from jax import lax
from jax.experimental import pallas as pl
from jax.experimental.pallas import tpu as pltpu
from jax.experimental.pallas import tpu_sc as plsc

EMBED = 64
BATCH = 16384
L = 16
NC, NS = 2, 16
NW = NC * NS
BPW = BATCH // NW
VOCAB_BLOCKS = 125000
NHALF = 8

_mesh = plsc.VectorSubcoreMesh(core_axis_name="c", subcore_axis_name="s")


@functools.partial(
    pl.kernel,
    out_type=jax.ShapeDtypeStruct((BATCH,), jnp.float32),
    mesh=_mesh,
    compiler_params=pltpu.CompilerParams(
        use_tc_tiling_on_sc=True, needs_layout_passes=False
    ),
    scratch_types=[
        pltpu.VMEM((BPW,), jnp.int32),
        pltpu.VMEM((BPW,), jnp.int32),
        pltpu.VMEM((NHALF, EMBED), jnp.float32),
        pltpu.VMEM((NHALF, EMBED), jnp.float32),
        pltpu.VMEM((BPW,), jnp.float32),
    ]
    + [pltpu.SemaphoreType.DMA] * NHALF
    + [pltpu.SemaphoreType.DMA] * NHALF,
)
def _skipgram_sc(cw_hbm, tw_hbm, ct_hbm, tt_hbm, out_hbm,
                 cidx, tidx, cr, tr, outv, *sems):
    csems = sems[:NHALF]
    tsems = sems[NHALF:]
    wid = lax.axis_index("s") * NC + lax.axis_index("c")
    base = wid * BPW

    pltpu.sync_copy(cw_hbm.at[pl.ds(base, BPW)], cidx)
    pltpu.sync_copy(tw_hbm.at[pl.ds(base, BPW)], tidx)

    def fire(c_row, t_row, slot, k):
        pltpu.async_copy(ct_hbm.at[c_row >> 3, c_row & 7],
                         cr.at[slot], csems[k])
        pltpu.async_copy(tt_hbm.at[t_row >> 3, t_row & 7],
                         tr.at[slot], tsems[k])

    def drain(slot, k):
        pltpu.make_async_copy(ct_hbm.at[0, 0], cr.at[slot], csems[k]).wait()
        pltpu.make_async_copy(tt_hbm.at[0, 0], tr.at[slot], tsems[k]).wait()

    lane = lax.iota(jnp.int32, L)

    civ0 = cidx[pl.ds(0, L)]
    tiv0 = tidx[pl.ds(0, L)]
    for k in range(NHALF):
        fire(civ0[k], tiv0[k], k, k)

    def compute(b, bi, acc):
        p = jnp.zeros((L,), jnp.float32)
        for c in range(EMBED // L):
            p = p + cr[b, pl.ds(c * L, L)] * tr[b, pl.ds(c * L, L)]
        dot = jnp.sum(p)
        return jnp.where(lane == bi, dot, acc)

    def block_steps(g, civ, tiv, nciv, ntiv, last):
        acc = jnp.zeros((L,), jnp.float32)
        for b in range(NHALF):
            drain(b, b)
            acc = compute(b, b, acc)
            fire(civ[b + NHALF], tiv[b + NHALF], b, b)
        for b in range(NHALF, L):
            drain(b - NHALF, b - NHALF)
            acc = compute(b - NHALF, b, acc)
            if not last:
                fire(nciv[b - NHALF], ntiv[b - NHALF], b - NHALF, b - NHALF)
        outv[pl.ds(g * L, L)] = acc

    def block_body(g, carry):
        civ = cidx[pl.ds(g * L, L)]
        tiv = tidx[pl.ds(g * L, L)]
        nciv = cidx[pl.ds((g + 1) * L, L)]
        ntiv = tidx[pl.ds((g + 1) * L, L)]
        block_steps(g, civ, tiv, nciv, ntiv, last=False)
        return carry

    n_blocks = BPW // L
    lax.fori_loop(0, n_blocks - 1, block_body, 0)

    g_last = n_blocks - 1
    civ = cidx[pl.ds(g_last * L, L)]
    tiv = tidx[pl.ds(g_last * L, L)]
    block_steps(g_last, civ, tiv, civ, tiv, last=True)

    pltpu.sync_copy(outv, out_hbm.at[pl.ds(base, BPW)])


def kernel(center_words, target_words, center_table, target_table):
    return _skipgram_sc(
        center_words.astype(jnp.int32),
        target_words.astype(jnp.int32),
        center_table.reshape(VOCAB_BLOCKS, 8, EMBED),
        target_table.reshape(VOCAB_BLOCKS, 8, EMBED),
    )
